# baseline (speedup 1.0000x reference)
import functools

import jax
import jax.numpy as jnp
from jax import lax
from jax.experimental import pallas as pl
from jax.experimental.pallas import tpu as pltpu
from jax.experimental.pallas import tpu_sc as plsc

NC = 2
NS = 16
NW = NC * NS
BC = 128
L = 4
TC_BLK = 512


def _sc_pool_kernel(n_units, n_idx_pairs, emb):
    upw = n_units // NW
    n_pairs = upw // 2
    assert n_pairs % 2 == 0
    owpu = BC * emb // 128
    mesh = plsc.VectorSubcoreMesh(
        core_axis_name="c", subcore_axis_name="s", num_cores=NC, num_subcores=NS
    )

    @functools.partial(
        pl.kernel,
        out_type=jax.ShapeDtypeStruct((n_units * owpu, 128), jnp.float32),
        mesh=mesh,
        scratch_types=[
            pltpu.VMEM((2 * L, BC), jnp.int32),
            pltpu.VMEM((2 * L, BC), jnp.int32),
            pltpu.VMEM((L, BC, emb), jnp.float32),
            pltpu.VMEM((L, BC, emb), jnp.float32),
            pltpu.VMEM((owpu, 128), jnp.float32),
            pltpu.VMEM((owpu, 128), jnp.float32),
            pltpu.SemaphoreType.DMA,
            pltpu.SemaphoreType.DMA,
        ],
        compiler_params=pltpu.CompilerParams(use_tc_tiling_on_sc=False),
    )
    def sc_kernel(idx_hbm, table_hbm, pooled_hbm,
                  idx_0, idx_1, r_a, r_b, out_a, out_b, sem_a, sem_b):
        wid = lax.axis_index("s") * NC + lax.axis_index("c")
        p_base = wid * n_pairs

        def fetch_idx(idx_v, p):
            pltpu.sync_copy(idx_hbm.at[p_base + p], idx_v)

        def fire(idx_v, half, r_v, sem):
            for l in range(L):
                pltpu.async_copy(
                    table_hbm.at[idx_v.at[half * L + l]], r_v.at[l], sem)

        def drain(idx_v, half, r_v, sem):
            for l in range(L):
                pltpu.make_async_copy(
                    table_hbm.at[idx_v.at[half * L + l]], r_v.at[l], sem).wait()

        def pool_write(r_v, out_v, p, half):
            def body(jw, carry):
                for q in range(4):
                    for h in range(emb // 16):
                        s = pl.ds(h * 16, 16)
                        j = 4 * jw + q
                        out_v[jw, pl.ds(32 * q + 16 * h, 16)] = (
                            (r_v[0, j, s] + r_v[1, j, s])
                            + (r_v[2, j, s] + r_v[3, j, s]))
                return carry

            lax.fori_loop(0, owpu, body, 0, unroll=2)
            u = (p_base + p) * 2 + half
            pltpu.sync_copy(out_v, pooled_hbm.at[pl.ds(u * owpu, owpu)])

        fetch_idx(idx_0, 0)
        fire(idx_0, 0, r_a, sem_a)
        fire(idx_0, 1, r_b, sem_b)

        def super_pair(t, carry):
            for s, idx_c, idx_n in ((0, idx_0, idx_1), (1, idx_1, idx_0)):
                p = 2 * t + s
                fetch_idx(idx_n, p + 1)
                drain(idx_c, 0, r_a, sem_a)
                pool_write(r_a, out_a, p, 0)
                fire(idx_n, 0, r_a, sem_a)
                drain(idx_c, 1, r_b, sem_b)
                pool_write(r_b, out_b, p, 1)
                fire(idx_n, 1, r_b, sem_b)
            return carry

        lax.fori_loop(0, n_pairs // 2, super_pair, 0)
        drain(idx_0, 0, r_a, sem_a)
        drain(idx_0, 1, r_b, sem_b)

    return sc_kernel


def _tc_proj_kernel(x_ref, w_ref, o_ref):
    res = jnp.dot(x_ref[...], w_ref[...], preferred_element_type=jnp.float32)
    o_ref[...] = res.reshape(o_ref.shape)


def _block_diag4(w_t):
    emb, hid = w_t.shape
    eye = jnp.eye(4, dtype=w_t.dtype)
    return (eye[:, None, :, None] * w_t[None, :, None, :]).reshape(4 * emb, 4 * hid)


def kernel(fields, cells, W_emb, W_fields, W_cells):
    B, K, Lf = fields.shape
    _, N, _, Lc = cells.shape
    assert Lf == L and Lc == L and B % BC == 0
    emb = W_emb.shape[1]
    hid = W_fields.shape[0]
    chunks = B // BC

    g_cells = N * K
    g_fields = K
    g_real = g_cells + g_fields
    g_unit = (4 * NW) // chunks if (4 * NW) % chunks == 0 else 4 * NW
    g_pad = ((g_real + g_unit - 1) // g_unit) * g_unit
    n_units = g_pad * chunks
    n_idx_pairs = n_units // 2 + 8
    g_idx = (2 * n_idx_pairs) // chunks

    cells_t = cells.astype(jnp.int32).transpose(1, 2, 3, 0)
    fields_t = fields.astype(jnp.int32).transpose(1, 2, 0)
    idx4 = jnp.concatenate([
        cells_t.reshape(g_cells, L, chunks, BC),
        fields_t.reshape(g_fields, L, chunks, BC),
        jnp.zeros((g_idx - g_real, L, chunks, BC), jnp.int32),
    ], axis=0)
    idx_pairs = idx4.transpose(0, 2, 1, 3).reshape(n_idx_pairs, 2 * L, BC)

    pooled = _sc_pool_kernel(n_units, n_idx_pairs, emb)(idx_pairs, W_emb)

    wc_big = _block_diag4((W_cells.T * (1.0 / L)).astype(jnp.float32))
    wf_big = _block_diag4((W_fields.T * (1.0 / L)).astype(jnp.float32))

    wide = 128 // emb
    gpb = TC_BLK * wide // B
    assert K % gpb == 0 and (g_cells * B // wide) % TC_BLK == 0

    out_c = pl.pallas_call(
        _tc_proj_kernel,
        grid=(g_cells // gpb,),
        in_specs=[
            pl.BlockSpec((TC_BLK, 128), lambda b: (b, 0)),
            pl.BlockSpec((128, wide * hid), lambda b: (0, 0)),
        ],
        out_specs=pl.BlockSpec((1, gpb, B, hid),
                               lambda b: (b // (K // gpb), b % (K // gpb), 0, 0)),
        out_shape=jax.ShapeDtypeStruct((N, K, B, hid), jnp.float32),
        compiler_params=pltpu.CompilerParams(
            dimension_semantics=("parallel",),
        ),
    )(pooled, wc_big)

    cells_blocks = g_cells // gpb

    out_f = pl.pallas_call(
        _tc_proj_kernel,
        grid=(g_fields // gpb,),
        in_specs=[
            pl.BlockSpec((TC_BLK, 128), lambda b: (b + cells_blocks, 0)),
            pl.BlockSpec((128, wide * hid), lambda b: (0, 0)),
        ],
        out_specs=pl.BlockSpec((gpb, B, hid), lambda b: (b, 0, 0)),
        out_shape=jax.ShapeDtypeStruct((K, B, hid), jnp.float32),
        compiler_params=pltpu.CompilerParams(
            dimension_semantics=("parallel",),
        ),
    )(pooled, wf_big)

    db_cells_out = out_c.transpose(2, 0, 1, 3)
    db_fields_out = out_f.transpose(1, 0, 2)
    return (db_fields_out, db_cells_out)


# speedup vs baseline: 1.0104x; 1.0104x over previous
import functools

import jax
import jax.numpy as jnp
from jax import lax
from jax.experimental import pallas as pl
from jax.experimental.pallas import tpu as pltpu
from jax.experimental.pallas import tpu_sc as plsc

NC = 2
NS = 16
NW = NC * NS
BC = 128
L = 4
TC_BLK = 512


def _sc_pool_kernel(n_units, n_idx_pairs, emb):
    upw = n_units // NW
    n_pairs = upw // 2
    assert n_pairs % 2 == 0
    owpu = BC * emb // 128
    mesh = plsc.VectorSubcoreMesh(
        core_axis_name="c", subcore_axis_name="s", num_cores=NC, num_subcores=NS
    )

    @functools.partial(
        pl.kernel,
        out_type=jax.ShapeDtypeStruct((n_units * owpu, 128), jnp.float32),
        mesh=mesh,
        scratch_types=[
            pltpu.VMEM((2 * L, BC), jnp.int32),
            pltpu.VMEM((2 * L, BC), jnp.int32),
            pltpu.VMEM((L, BC, emb), jnp.float32),
            pltpu.VMEM((L, BC, emb), jnp.float32),
            pltpu.VMEM((2, owpu, 128), jnp.float32),
            pltpu.VMEM((2, owpu, 128), jnp.float32),
            pltpu.SemaphoreType.DMA,
            pltpu.SemaphoreType.DMA,
            pltpu.SemaphoreType.DMA,
            pltpu.SemaphoreType.DMA,
            pltpu.SemaphoreType.DMA,
            pltpu.SemaphoreType.DMA,
            pltpu.SemaphoreType.DMA,
            pltpu.SemaphoreType.DMA,
        ],
        compiler_params=pltpu.CompilerParams(use_tc_tiling_on_sc=False),
    )
    def sc_kernel(idx_hbm, table_hbm, pooled_hbm,
                  idx_0, idx_1, r_a, r_b, out_a, out_b,
                  sem_a, sem_b, sem_i0, sem_i1,
                  sem_wa0, sem_wa1, sem_wb0, sem_wb1):
        wid = lax.axis_index("s") * NC + lax.axis_index("c")
        p_base = wid * n_pairs
        idx_sems = (sem_i0, sem_i1)
        w_sems = ((sem_wa0, sem_wa1), (sem_wb0, sem_wb1))
        outs = (out_a, out_b)

        def fire(idx_v, half, r_v, sem):
            for l in range(L):
                pltpu.async_copy(
                    table_hbm.at[idx_v.at[half * L + l]], r_v.at[l], sem)

        def drain(idx_v, half, r_v, sem):
            for l in range(L):
                pltpu.make_async_copy(
                    table_hbm.at[idx_v.at[half * L + l]], r_v.at[l], sem).wait()

        def wait_write(half, par, p):
            out_v = outs[half].at[par]
            u = (p_base + p) * 2 + half
            pltpu.make_async_copy(
                out_v, pooled_hbm.at[pl.ds(u * owpu, owpu)], w_sems[half][par]
            ).wait()

        def pool_write(r_v, half, par, p):
            out_v = outs[half].at[par]

            def body(jw, carry):
                for q in range(4):
                    for h in range(emb // 16):
                        s = pl.ds(h * 16, 16)
                        j = 4 * jw + q
                        out_v[jw, pl.ds(32 * q + 16 * h, 16)] = (
                            (r_v[0, j, s] + r_v[1, j, s])
                            + (r_v[2, j, s] + r_v[3, j, s]))
                return carry

            lax.fori_loop(0, owpu, body, 0, unroll=4)
            u = (p_base + p) * 2 + half
            pltpu.async_copy(
                out_v, pooled_hbm.at[pl.ds(u * owpu, owpu)], w_sems[half][par])

        pltpu.sync_copy(idx_hbm.at[p_base], idx_0)
        fire(idx_0, 0, r_a, sem_a)
        fire(idx_0, 1, r_b, sem_b)
        pltpu.async_copy(idx_hbm.at[p_base + 1], idx_1, sem_i1)

        def super_pair(t, carry):
            for s, idx_c, idx_n in ((0, idx_0, idx_1), (1, idx_1, idx_0)):
                p = 2 * t + s
                pltpu.make_async_copy(
                    idx_hbm.at[p_base + p + 1], idx_n, idx_sems[1 - s]).wait()
                drain(idx_c, 0, r_a, sem_a)

                @pl.when(t > 0)
                def _():
                    wait_write(0, s, p - 2)

                pool_write(r_a, 0, s, p)
                fire(idx_n, 0, r_a, sem_a)
                drain(idx_c, 1, r_b, sem_b)
                pltpu.async_copy(
                    idx_hbm.at[p_base + p + 2], idx_c, idx_sems[s])

                @pl.when(t > 0)
                def _():
                    wait_write(1, s, p - 2)

                pool_write(r_b, 1, s, p)
                fire(idx_n, 1, r_b, sem_b)
            return carry

        lax.fori_loop(0, n_pairs // 2, super_pair, 0)
        drain(idx_0, 0, r_a, sem_a)
        drain(idx_0, 1, r_b, sem_b)
        pltpu.make_async_copy(idx_hbm.at[p_base + n_pairs + 1], idx_1,
                              idx_sems[1]).wait()
        for half in (0, 1):
            for par in (0, 1):
                wait_write(half, par, n_pairs - 2 + par)

    return sc_kernel


def _tc_proj_kernel(x_ref, w_ref, o_ref):
    res = jnp.dot(x_ref[...], w_ref[...], preferred_element_type=jnp.float32)
    o_ref[...] = res.reshape(o_ref.shape)


def _block_diag4(w_t):
    emb, hid = w_t.shape
    eye = jnp.eye(4, dtype=w_t.dtype)
    return (eye[:, None, :, None] * w_t[None, :, None, :]).reshape(4 * emb, 4 * hid)


def kernel(fields, cells, W_emb, W_fields, W_cells):
    B, K, Lf = fields.shape
    _, N, _, Lc = cells.shape
    assert Lf == L and Lc == L and B % BC == 0
    emb = W_emb.shape[1]
    hid = W_fields.shape[0]
    chunks = B // BC

    g_cells = N * K
    g_fields = K
    g_real = g_cells + g_fields
    g_unit = (4 * NW) // chunks if (4 * NW) % chunks == 0 else 4 * NW
    g_pad = ((g_real + g_unit - 1) // g_unit) * g_unit
    n_units = g_pad * chunks
    n_idx_pairs = n_units // 2 + 8
    g_idx = (2 * n_idx_pairs) // chunks

    cells_t = cells.astype(jnp.int32).transpose(1, 2, 3, 0)
    fields_t = fields.astype(jnp.int32).transpose(1, 2, 0)
    idx4 = jnp.concatenate([
        cells_t.reshape(g_cells, L, chunks, BC),
        fields_t.reshape(g_fields, L, chunks, BC),
        jnp.zeros((g_idx - g_real, L, chunks, BC), jnp.int32),
    ], axis=0)
    idx_pairs = idx4.transpose(0, 2, 1, 3).reshape(n_idx_pairs, 2 * L, BC)

    pooled = _sc_pool_kernel(n_units, n_idx_pairs, emb)(idx_pairs, W_emb)

    wc_big = _block_diag4((W_cells.T * (1.0 / L)).astype(jnp.float32))
    wf_big = _block_diag4((W_fields.T * (1.0 / L)).astype(jnp.float32))

    wide = 128 // emb
    gpb = TC_BLK * wide // B
    assert K % gpb == 0 and (g_cells * B // wide) % TC_BLK == 0

    out_c = pl.pallas_call(
        _tc_proj_kernel,
        grid=(g_cells // gpb,),
        in_specs=[
            pl.BlockSpec((TC_BLK, 128), lambda b: (b, 0)),
            pl.BlockSpec((128, wide * hid), lambda b: (0, 0)),
        ],
        out_specs=pl.BlockSpec((1, gpb, B, hid),
                               lambda b: (b // (K // gpb), b % (K // gpb), 0, 0)),
        out_shape=jax.ShapeDtypeStruct((N, K, B, hid), jnp.float32),
        compiler_params=pltpu.CompilerParams(
            dimension_semantics=("parallel",),
        ),
    )(pooled, wc_big)

    cells_blocks = g_cells // gpb

    out_f = pl.pallas_call(
        _tc_proj_kernel,
        grid=(g_fields // gpb,),
        in_specs=[
            pl.BlockSpec((TC_BLK, 128), lambda b: (b + cells_blocks, 0)),
            pl.BlockSpec((128, wide * hid), lambda b: (0, 0)),
        ],
        out_specs=pl.BlockSpec((gpb, B, hid), lambda b: (b, 0, 0)),
        out_shape=jax.ShapeDtypeStruct((K, B, hid), jnp.float32),
        compiler_params=pltpu.CompilerParams(
            dimension_semantics=("parallel",),
        ),
    )(pooled, wf_big)

    db_cells_out = out_c.transpose(2, 0, 1, 3)
    db_fields_out = out_f.transpose(1, 0, 2)
    return (db_fields_out, db_cells_out)


# speedup vs baseline: 1.0225x; 1.0119x over previous
import functools

import jax
import jax.numpy as jnp
from jax import lax
from jax.experimental import pallas as pl
from jax.experimental.pallas import tpu as pltpu
from jax.experimental.pallas import tpu_sc as plsc

NC = 2
NS = 16
NW = NC * NS
BC = 128
L = 4
TC_BLK = 512


def _sc_pool_kernel(n_units, n_idx_pairs, emb):
    upw = n_units // NW
    n_pairs = upw // 2
    assert n_pairs % 2 == 0
    owpu = BC * emb // 128
    mesh = plsc.VectorSubcoreMesh(
        core_axis_name="c", subcore_axis_name="s", num_cores=NC, num_subcores=NS
    )

    @functools.partial(
        pl.kernel,
        out_type=jax.ShapeDtypeStruct((n_units * owpu, 128), jnp.float32),
        mesh=mesh,
        scratch_types=[
            pltpu.VMEM((2 * L, BC), jnp.int32),
            pltpu.VMEM((2 * L, BC), jnp.int32),
            pltpu.VMEM((L, BC, emb), jnp.float32),
            pltpu.VMEM((L, BC, emb), jnp.float32),
            pltpu.VMEM((2, owpu, 128), jnp.float32),
            pltpu.VMEM((2, owpu, 128), jnp.float32),
            pltpu.SemaphoreType.DMA,
            pltpu.SemaphoreType.DMA,
            pltpu.SemaphoreType.DMA,
            pltpu.SemaphoreType.DMA,
            pltpu.SemaphoreType.DMA,
            pltpu.SemaphoreType.DMA,
            pltpu.SemaphoreType.DMA,
            pltpu.SemaphoreType.DMA,
        ],
        compiler_params=pltpu.CompilerParams(use_tc_tiling_on_sc=False),
    )
    def sc_kernel(idx_hbm, table_hbm, pooled_hbm,
                  idx_0, idx_1, r_a, r_b, out_a, out_b,
                  sem_a, sem_b, sem_i0, sem_i1,
                  sem_wa0, sem_wa1, sem_wb0, sem_wb1):
        wid = lax.axis_index("s") * NC + lax.axis_index("c")
        p_base = wid * n_pairs
        idx_sems = (sem_i0, sem_i1)
        w_sems = ((sem_wa0, sem_wa1), (sem_wb0, sem_wb1))
        outs = (out_a, out_b)

        def fire(idx_v, half, r_v, sem):
            for l in range(L):
                pltpu.async_copy(
                    table_hbm.at[idx_v.at[half * L + l]], r_v.at[l], sem)

        def drain(idx_v, half, r_v, sem):
            for l in range(L):
                pltpu.make_async_copy(
                    table_hbm.at[idx_v.at[half * L + l]], r_v.at[l], sem).wait()

        def wait_write(half, par, p):
            out_v = outs[half].at[par]
            u = (p_base + p) * 2 + half
            pltpu.make_async_copy(
                out_v, pooled_hbm.at[pl.ds(u * owpu, owpu)], w_sems[half][par]
            ).wait()

        def pool_write(r_v, half, par, p):
            out_v = outs[half].at[par]

            @plsc.parallel_loop(0, owpu, unroll=4)
            def _(jw):
                for q in range(4):
                    for h in range(emb // 16):
                        s = pl.ds(h * 16, 16)
                        j = 4 * jw + q
                        out_v[jw, pl.ds(32 * q + 16 * h, 16)] = (
                            (r_v[0, j, s] + r_v[1, j, s])
                            + (r_v[2, j, s] + r_v[3, j, s]))
            u = (p_base + p) * 2 + half
            pltpu.async_copy(
                out_v, pooled_hbm.at[pl.ds(u * owpu, owpu)], w_sems[half][par])

        pltpu.sync_copy(idx_hbm.at[p_base], idx_0)
        fire(idx_0, 0, r_a, sem_a)
        fire(idx_0, 1, r_b, sem_b)
        pltpu.async_copy(idx_hbm.at[p_base + 1], idx_1, sem_i1)

        def super_pair(t, carry):
            for s, idx_c, idx_n in ((0, idx_0, idx_1), (1, idx_1, idx_0)):
                p = 2 * t + s
                pltpu.make_async_copy(
                    idx_hbm.at[p_base + p + 1], idx_n, idx_sems[1 - s]).wait()
                drain(idx_c, 0, r_a, sem_a)

                @pl.when(t > 0)
                def _():
                    wait_write(0, s, p - 2)

                pool_write(r_a, 0, s, p)
                fire(idx_n, 0, r_a, sem_a)
                drain(idx_c, 1, r_b, sem_b)
                pltpu.async_copy(
                    idx_hbm.at[p_base + p + 2], idx_c, idx_sems[s])

                @pl.when(t > 0)
                def _():
                    wait_write(1, s, p - 2)

                pool_write(r_b, 1, s, p)
                fire(idx_n, 1, r_b, sem_b)
            return carry

        lax.fori_loop(0, n_pairs // 2, super_pair, 0)
        drain(idx_0, 0, r_a, sem_a)
        drain(idx_0, 1, r_b, sem_b)
        pltpu.make_async_copy(idx_hbm.at[p_base + n_pairs + 1], idx_1,
                              idx_sems[1]).wait()
        for half in (0, 1):
            for par in (0, 1):
                wait_write(half, par, n_pairs - 2 + par)

    return sc_kernel


def _tc_proj_kernel(x_ref, w_ref, o_ref):
    res = jnp.dot(x_ref[...], w_ref[...], preferred_element_type=jnp.float32)
    o_ref[...] = res.reshape(o_ref.shape)


def _block_diag4(w_t):
    emb, hid = w_t.shape
    eye = jnp.eye(4, dtype=w_t.dtype)
    return (eye[:, None, :, None] * w_t[None, :, None, :]).reshape(4 * emb, 4 * hid)


def kernel(fields, cells, W_emb, W_fields, W_cells):
    B, K, Lf = fields.shape
    _, N, _, Lc = cells.shape
    assert Lf == L and Lc == L and B % BC == 0
    emb = W_emb.shape[1]
    hid = W_fields.shape[0]
    chunks = B // BC

    g_cells = N * K
    g_fields = K
    g_real = g_cells + g_fields
    g_unit = (4 * NW) // chunks if (4 * NW) % chunks == 0 else 4 * NW
    g_pad = ((g_real + g_unit - 1) // g_unit) * g_unit
    n_units = g_pad * chunks
    n_idx_pairs = n_units // 2 + 8
    g_idx = (2 * n_idx_pairs) // chunks

    cells_t = cells.astype(jnp.int32).transpose(1, 2, 3, 0)
    fields_t = fields.astype(jnp.int32).transpose(1, 2, 0)
    idx4 = jnp.concatenate([
        cells_t.reshape(g_cells, L, chunks, BC),
        fields_t.reshape(g_fields, L, chunks, BC),
        jnp.zeros((g_idx - g_real, L, chunks, BC), jnp.int32),
    ], axis=0)
    idx_pairs = idx4.transpose(0, 2, 1, 3).reshape(n_idx_pairs, 2 * L, BC)

    pooled = _sc_pool_kernel(n_units, n_idx_pairs, emb)(idx_pairs, W_emb)

    wc_big = _block_diag4((W_cells.T * (1.0 / L)).astype(jnp.float32))
    wf_big = _block_diag4((W_fields.T * (1.0 / L)).astype(jnp.float32))

    wide = 128 // emb
    gpb = TC_BLK * wide // B
    assert K % gpb == 0 and (g_cells * B // wide) % TC_BLK == 0

    out_c = pl.pallas_call(
        _tc_proj_kernel,
        grid=(g_cells // gpb,),
        in_specs=[
            pl.BlockSpec((TC_BLK, 128), lambda b: (b, 0)),
            pl.BlockSpec((128, wide * hid), lambda b: (0, 0)),
        ],
        out_specs=pl.BlockSpec((1, gpb, B, hid),
                               lambda b: (b // (K // gpb), b % (K // gpb), 0, 0)),
        out_shape=jax.ShapeDtypeStruct((N, K, B, hid), jnp.float32),
        compiler_params=pltpu.CompilerParams(
            dimension_semantics=("parallel",),
        ),
    )(pooled, wc_big)

    cells_blocks = g_cells // gpb

    out_f = pl.pallas_call(
        _tc_proj_kernel,
        grid=(g_fields // gpb,),
        in_specs=[
            pl.BlockSpec((TC_BLK, 128), lambda b: (b + cells_blocks, 0)),
            pl.BlockSpec((128, wide * hid), lambda b: (0, 0)),
        ],
        out_specs=pl.BlockSpec((gpb, B, hid), lambda b: (b, 0, 0)),
        out_shape=jax.ShapeDtypeStruct((K, B, hid), jnp.float32),
        compiler_params=pltpu.CompilerParams(
            dimension_semantics=("parallel",),
        ),
    )(pooled, wf_big)

    db_cells_out = out_c.transpose(2, 0, 1, 3)
    db_fields_out = out_f.transpose(1, 0, 2)
    return (db_fields_out, db_cells_out)
